# baseline (speedup 1.0000x reference)
import functools

import jax
import jax.numpy as jnp
from jax import lax
from jax.experimental import pallas as pl
from jax.experimental.pallas import tpu as pltpu
from jax.experimental.pallas import tpu_sc as plsc

_N, _E, _D, _R = 10000, 160000, 256, 8
_TN = 400
_NT = _N // _TN
_TNA = 2000
_NTA = _N // _TNA
_EP = 10240
_NSUB = 16
_EPAD = _EP * _NSUB
_CB = 32
_BLK = 256
_NBLK = _EP // _BLK
_NROW = 10112
_RPS = _NROW // _NSUB
_DH = _D // 2
_NH = 5056
_AGGR = 5120
_RPH = _AGGR // _NSUB


def _proj_body(x_ref, xbf_ref, w_ref, p_ref, sw_ref, sb_ref, xw_ref,
               sprj_ref, so_ref, st_ref):
    n = pl.program_id(0)
    r = pl.program_id(1)
    xb = x_ref[...]
    xw_ref[0] = jnp.dot(xbf_ref[...], w_ref[0],
                        preferred_element_type=jnp.float32)

    @pl.when(r == 0)
    def _():
        xp = jnp.dot(xb, p_ref[...], preferred_element_type=jnp.float32)
        sprj_ref[...] = xp
        so_ref[...] = (
            jnp.dot(xb, sw_ref[...], preferred_element_type=jnp.float32)
            + sb_ref[...]
        )
        bm = jnp.max(xp, axis=0, keepdims=True)
        st_ref[...] = jnp.where(n == 0, bm, jnp.maximum(st_ref[...], bm))


def _run_proj(x, xbf, weight, p, self_w, self_b2):
    return pl.pallas_call(
        _proj_body,
        grid=(_NTA, _R),
        in_specs=[
            pl.BlockSpec((_TNA, _D), lambda n, r: (n, 0)),
            pl.BlockSpec((_TNA, _D), lambda n, r: (n, 0)),
            pl.BlockSpec((1, _D, _D), lambda n, r: (r, 0, 0)),
            pl.BlockSpec((_D, 8), lambda n, r: (0, 0)),
            pl.BlockSpec((_D, _D), lambda n, r: (0, 0)),
            pl.BlockSpec((1, _D), lambda n, r: (0, 0)),
        ],
        out_specs=[
            pl.BlockSpec((1, _TNA, _D), lambda n, r: (r, n, 0)),
            pl.BlockSpec((_TNA, 8), lambda n, r: (n, 0)),
            pl.BlockSpec((_TNA, _D), lambda n, r: (n, 0)),
            pl.BlockSpec((1, 8), lambda n, r: (0, 0)),
        ],
        out_shape=[
            jax.ShapeDtypeStruct((_R, _N, _D), jnp.float32),
            jax.ShapeDtypeStruct((_N, 8), jnp.float32),
            jax.ShapeDtypeStruct((_N, _D), jnp.float32),
            jax.ShapeDtypeStruct((1, 8), jnp.float32),
        ],
    )(x, xbf, weight, p, self_w, self_b2)


_sc_mesh = plsc.VectorSubcoreMesh(core_axis_name="c", subcore_axis_name="s")


@functools.partial(
    pl.kernel,
    out_type=jax.ShapeDtypeStruct((2, 2, _AGGR, _DH), jnp.float32),
    mesh=_sc_mesh,
    compiler_params=pltpu.CompilerParams(needs_layout_passes=False),
    scratch_types=[
        pltpu.VMEM((_NROW,), jnp.float32),
        pltpu.VMEM((_NROW,), jnp.float32),
        pltpu.VMEM((128, 128), jnp.float32),
        pltpu.VMEM((4 * _BLK,), jnp.int32),
        pltpu.VMEM((_CB, _DH), jnp.float32),
        pltpu.VMEM((_CB, _DH), jnp.float32),
        pltpu.VMEM((_CB,), jnp.int32),
        pltpu.VMEM((_CB,), jnp.int32),
        pltpu.VMEM((_CB,), jnp.int32),
        pltpu.VMEM((_CB,), jnp.int32),
        pltpu.VMEM((_CB,), jnp.float32),
        pltpu.VMEM((_CB,), jnp.float32),
        pltpu.VMEM((16,), jnp.float32),
        pltpu.VMEM((1, 128), jnp.int32),
        pltpu.VMEM((2 * _NBLK + 16,), jnp.int32),
        pltpu.VMEM_SHARED((128, 128), jnp.float32),
        pltpu.VMEM_SHARED((_AGGR, _DH), jnp.float32),
        pltpu.SemaphoreType.DMA,
        pltpu.SemaphoreType.DMA,
        pltpu.SemaphoreType.DMA,
        pltpu.SemaphoreType.DMA,
    ],
)
def _sc_agg(s1_h, s2_h, ed_h, mh_h, fl_h, xw_h, out_h,
            s1_v, s2_v, esum_v, edA,
            rows_v0, rows_v1, gidx_v0, gidx_v1, sidx_v0, sidx_v1,
            coef_v0, coef_v1, mh_v, ridx_v, flag_v,
            esum_sh, agg_sh, gsem0, gsem1, ssem0, ssem1):
    c = lax.axis_index("c")
    s = lax.axis_index("s")
    zero16 = jnp.zeros((16,), jnp.float32)

    def _load_blk(b):
        pltpu.sync_copy(ed_h.at[pl.ds((s * _NBLK + b) * 4 * _BLK, 4 * _BLK)],
                        edA)

    def _srcdst(sl16):
        src16 = edA[pl.ds(sl16, 16)]
        dst16 = edA[pl.ds(_BLK + sl16, 16)]
        return src16, dst16

    def _zt(i, _):
        sl = pl.ds(i * 16, 16)
        s1_v[sl] = zero16
        s2_v[sl] = zero16
        return 0
    lax.fori_loop(0, _NROW // 16, _zt, 0)

    def _ze(i, _):
        for v in range(8):
            esum_v[i, pl.ds(v * 16, 16)] = zero16
        return 0
    lax.fori_loop(0, 128, _ze, 0)

    pltpu.sync_copy(s1_h, s1_v.at[pl.ds(0, _N)])
    pltpu.sync_copy(s2_h, s2_v.at[pl.ds(0, _N)])
    pltpu.sync_copy(mh_h, mh_v)
    pltpu.sync_copy(fl_h.at[s], flag_v)

    for v in range(8):
        ridx_v[0, pl.ds(v * 16, 16)] = lax.iota(jnp.int32, 16) + v * 16

    def _zero_rows():
        def _zr(e, _):
            for v in range(_DH // 16):
                rows_v0[e, pl.ds(v * 16, 16)] = zero16
            return 0
        lax.fori_loop(0, _CB, _zr, 0)

    _zero_rows()

    r0 = s * _RPH

    def _zero_agg():
        for i in range(_RPH // _CB):
            pltpu.sync_copy(rows_v0, agg_sh.at[pl.ds(r0 + i * _CB, _CB)])

    @pl.when(s == 0)
    def _():
        pltpu.sync_copy(esum_v, esum_sh)

    plsc.subcore_barrier()

    mhat = mh_v[...]

    def _escore(src16, dst16):
        e = plsc.load_gather(s1_v, [src16]) + plsc.load_gather(s2_v, [dst16])
        e = jnp.where(e > 0, e, e * 0.01)
        return jnp.exp(e - mhat)

    def _esum_idx(dst16):
        return [lax.shift_right_logical(dst16, 7), jnp.bitwise_and(dst16, 127)]

    def _pa_blk(b, _):
        _load_blk(b)

        def _pa(i, _):
            src16, dst16 = _srcdst(i * 16)
            ex = _escore(src16, dst16)
            plsc.addupdate_scatter(esum_v, _esum_idx(dst16), ex)
            return 0
        lax.fori_loop(0, _BLK // 16, _pa, 0)
        return 0
    lax.fori_loop(0, _NBLK, _pa_blk, 0)

    pltpu.sync_copy(esum_v, esum_sh.at[ridx_v.at[0]], add=True)
    plsc.subcore_barrier()
    pltpu.sync_copy(esum_sh, esum_v)

    for h in range(2):
        _zero_rows()
        _zero_agg()
        plsc.subcore_barrier()

        bufs = ((rows_v0, gidx_v0, sidx_v0, coef_v0, gsem0, ssem0),
                (rows_v1, gidx_v1, sidx_v1, coef_v1, gsem1, ssem1))
        nch = _BLK // _CB

        def _build(u, st):
            rows, gidx, sidx, coef, gsem, ssem = st
            for v in range(_CB // 16):
                sl16 = u * _CB + v * 16
                osl = pl.ds(v * 16, 16)
                src16, dst16 = _srcdst(sl16)
                ex = _escore(src16, dst16)
                es = plsc.load_gather(esum_v, _esum_idx(dst16))
                nw16 = plsc.bitcast(edA[pl.ds(3 * _BLK + sl16, 16)],
                                    jnp.float32)
                cf = ex * nw16 / (es + 1e-9)
                t16 = dst16 - h * _NH
                inr = jnp.logical_and(t16 >= 0, t16 < _NH)
                coef[osl] = jnp.where(inr, cf, 0.0)
                sidx[osl] = jnp.where(inr, t16, _NH)
                gidx[osl] = edA[pl.ds(2 * _BLK + sl16, 16)] + c

        def _scale(st):
            rows, gidx, sidx, coef, gsem, ssem = st

            def _sc4(i4, _):
                for k in range(4):
                    e2 = i4 * 4 + k
                    cvec = plsc.load_gather(
                        coef, [jnp.zeros((16,), jnp.int32) + e2])
                    for v in range(_DH // 16):
                        csl = pl.ds(v * 16, 16)
                        rows[e2, csl] = rows[e2, csl] * cvec
                return 0
            lax.fori_loop(0, _CB // 4, _sc4, 0)

        def _pb_blk(b, _):
            fv = flag_v[pl.ds(h * _NBLK + b, 16)]

            @pl.when(fv[0] > 0)
            def _():
                _pb_blk_body(b)
            return 0

        def _pb_blk_body(b):
            _load_blk(b)

            _build(0, bufs[0])
            gd = {0: pltpu.async_copy(
                xw_h.at[bufs[0][1]], bufs[0][0], bufs[0][4])}
            sd = {}
            for u in range(nch):
                cur = bufs[u % 2]
                nxt = bufs[(u + 1) % 2]
                if u + 1 < nch:
                    if u >= 1:
                        sd[u - 1].wait()
                    _build(u + 1, nxt)
                    gd[u + 1] = pltpu.async_copy(
                        xw_h.at[nxt[1]], nxt[0], nxt[4])
                gd[u].wait()
                _scale(cur)
                sd[u] = pltpu.async_copy(
                    cur[0], agg_sh.at[cur[2]], cur[5], add=True)
            sd[nch - 2].wait()
            sd[nch - 1].wait()
        lax.fori_loop(0, _NBLK, _pb_blk, 0)

        plsc.subcore_barrier()
        pltpu.sync_copy(agg_sh.at[pl.ds(r0, _RPH)],
                        out_h.at[c, h, pl.ds(r0, _RPH)])
        plsc.subcore_barrier()


def _sg_body(q_ref, k_ref, ti_ref, tj_ref, xj_ref, g1_ref, so_ref,
             pw_ref, pb_ref, g2_ref, out_ref, num_ref, den_ref):
    j = pl.program_id(1)

    @pl.when(j == 0)
    def _():
        num_ref[...] = jnp.zeros_like(num_ref)
        den_ref[...] = jnp.zeros_like(den_ref)

    sc = q_ref[...] + k_ref[0]
    sc = jnp.where(sc > 0, sc, sc * 0.01)
    allowed = ti_ref[...] != tj_ref[0]
    w = jnp.where(allowed, jnp.exp(sc), 0.0)
    num_ref[...] += jnp.dot(w.astype(jnp.bfloat16), xj_ref[...],
                            preferred_element_type=jnp.float32)
    den_ref[...] += jnp.sum(w, axis=1, keepdims=True)

    @pl.when(j == _NT - 1)
    def _():
        sg_msg = num_ref[...] / den_ref[...]
        gate_in = g1_ref[...] + jnp.dot(
            sg_msg, g2_ref[...], preferred_element_type=jnp.float32)
        gate = 1.0 / (1.0 + jnp.exp(-gate_in))
        sg_out = (jnp.dot(sg_msg, pw_ref[...], preferred_element_type=jnp.float32)
                  + pb_ref[...]) * gate
        out_ref[...] = so_ref[...] + sg_out


def _run_sg(q2, k2, ti2, tj2, x, g12, so, pw, pb2, g2w):
    return pl.pallas_call(
        _sg_body,
        grid=(_NT, _NT),
        in_specs=[
            pl.BlockSpec((_TN, 1), lambda i, j: (i, 0)),
            pl.BlockSpec((1, 1, _TN), lambda i, j: (j, 0, 0)),
            pl.BlockSpec((_TN, 1), lambda i, j: (i, 0)),
            pl.BlockSpec((1, 1, _TN), lambda i, j: (j, 0, 0)),
            pl.BlockSpec((_TN, _D), lambda i, j: (j, 0)),
            pl.BlockSpec((_TN, 1), lambda i, j: (i, 0)),
            pl.BlockSpec((_TN, _D), lambda i, j: (i, 0)),
            pl.BlockSpec((_D, _D), lambda i, j: (0, 0)),
            pl.BlockSpec((1, _D), lambda i, j: (0, 0)),
            pl.BlockSpec((_D, 1), lambda i, j: (0, 0)),
        ],
        out_specs=pl.BlockSpec((_TN, _D), lambda i, j: (i, 0)),
        out_shape=jax.ShapeDtypeStruct((_N, _D), jnp.float32),
        scratch_shapes=[
            pltpu.VMEM((_TN, _D), jnp.float32),
            pltpu.VMEM((_TN, 1), jnp.float32),
        ],
    )(q2, k2, ti2, tj2, x, g12, so, pw, pb2, g2w)


def kernel(x, edge_index, rel_type, subgraph_type, norm, weight, attn_w,
           subgraph_attn_w, sg_proj_w, sg_proj_b, sg_gate_w, self_w, self_b):
    n, d = x.shape

    p = jnp.concatenate(
        [attn_w[:d], attn_w[d:], subgraph_attn_w[:d], subgraph_attn_w[d:],
         sg_gate_w[:d], jnp.zeros((d, 3), jnp.float32)], axis=1)

    xbf = x.astype(jnp.bfloat16)
    xw, sprj, selfout, stats = _run_proj(
        x, xbf, weight.astype(jnp.bfloat16), p, self_w, self_b.reshape(1, d))
    mh16 = jnp.full((16,), jnp.maximum(stats[0, 0] + stats[0, 1], 0.0),
                    jnp.float32)
    s1 = sprj[:, 0]
    s2 = sprj[:, 1]
    kk = sprj[:, 2]
    qq = sprj[:, 3]
    g1 = sprj[:, 4]

    src = edge_index[0]
    dst = edge_index[1]
    pad = _EPAD - _E
    srcp = jnp.concatenate([src, jnp.zeros((pad,), jnp.int32)])
    dstp = jnp.concatenate([dst, jnp.full((pad,), n, jnp.int32)])
    gb2 = (rel_type * n + src) * 2
    gbp = jnp.concatenate([gb2, jnp.zeros((pad,), jnp.int32)])
    nwp = jnp.concatenate([norm, jnp.zeros((pad,), jnp.float32)])

    half = dstp >= _NH
    c0 = jnp.cumsum(jnp.where(half, 0, 1))
    c1 = jnp.cumsum(jnp.where(half, 1, 0))
    pos = jnp.where(half, c0[-1] + c1, c0) - 1
    blk = pos // _BLK
    pbase = (((blk % _NSUB) * _NBLK + blk // _NSUB) * 4 * _BLK
             + pos % _BLK)
    ed = jnp.zeros((_EPAD * 4,), jnp.int32)
    for off, arr in ((0, srcp), (_BLK, dstp), (2 * _BLK, gbp),
                     (3 * _BLK, lax.bitcast_convert_type(nwp, jnp.int32))):
        ed = ed.at[pbase + off].set(
            arr, mode="promise_in_bounds", unique_indices=True)
    ed4 = ed.reshape(_NSUB, _NBLK, 4, _BLK)
    dblk = ed4[:, :, 1, :]
    live = ed4[:, :, 3, :] != 0
    fl0 = jnp.any((dblk < _NH) & live, axis=2)
    fl1 = jnp.any((dblk >= _NH) & live, axis=2)
    flags = jnp.concatenate(
        [fl0, fl1, jnp.zeros((_NSUB, 16), bool)], axis=1).astype(jnp.int32)
    xwf = xw.reshape(_R * n * 2, _DH)

    agg4 = _sc_agg(s1, s2, ed, mh16, flags, xwf)

    sgso = _run_sg(
        qq.reshape(n, 1), kk.reshape(_NT, 1, _TN),
        subgraph_type.reshape(n, 1), subgraph_type.reshape(_NT, 1, _TN),
        xbf, g1.reshape(n, 1), selfout,
        sg_proj_w, sg_proj_b.reshape(1, d), sg_gate_w[d:])

    agg = jnp.concatenate(
        [jnp.concatenate([agg4[0, 0, :_NH], agg4[1, 0, :_NH]], axis=1),
         jnp.concatenate([agg4[0, 1, :_NH], agg4[1, 1, :_NH]], axis=1)],
        axis=0)[:n]
    return sgso + agg


# speedup vs baseline: 3.2287x; 3.2287x over previous
import functools

import jax
import jax.numpy as jnp
from jax import lax
from jax.experimental import pallas as pl
from jax.experimental.pallas import tpu as pltpu
from jax.experimental.pallas import tpu_sc as plsc

_N, _E, _D, _R = 10000, 160000, 256, 8
_TN = 400
_NT = _N // _TN
_TNA = 2000
_NTA = _N // _TNA
_EP = 10240
_NSUB = 16
_EPAD = _EP * _NSUB
_CB = 32
_BLK = 256
_NBLK = _EP // _BLK
_NROW = 10112
_RPS = _NROW // _NSUB
_DH = _D // 2
_NH = 5056
_AGGR = 5120
_RPH = _AGGR // _NSUB


def _proj_body(x_ref, xbf_ref, w_ref, p_ref, sw_ref, sb_ref, xw_ref,
               sprj_ref, so_ref, st_ref):
    n = pl.program_id(0)
    r = pl.program_id(1)
    xb = x_ref[...]
    xw_ref[0] = jnp.dot(xbf_ref[...], w_ref[0],
                        preferred_element_type=jnp.float32)

    @pl.when(r == 0)
    def _():
        xp = jnp.dot(xb, p_ref[...], preferred_element_type=jnp.float32)
        sprj_ref[...] = xp
        so_ref[...] = (
            jnp.dot(xb, sw_ref[...], preferred_element_type=jnp.float32)
            + sb_ref[...]
        )
        bm = jnp.max(xp, axis=0, keepdims=True)
        st_ref[...] = jnp.where(n == 0, bm, jnp.maximum(st_ref[...], bm))


def _run_proj(x, xbf, weight, p, self_w, self_b2):
    return pl.pallas_call(
        _proj_body,
        grid=(_NTA, _R),
        in_specs=[
            pl.BlockSpec((_TNA, _D), lambda n, r: (n, 0)),
            pl.BlockSpec((_TNA, _D), lambda n, r: (n, 0)),
            pl.BlockSpec((1, _D, _D), lambda n, r: (r, 0, 0)),
            pl.BlockSpec((_D, 8), lambda n, r: (0, 0)),
            pl.BlockSpec((_D, _D), lambda n, r: (0, 0)),
            pl.BlockSpec((1, _D), lambda n, r: (0, 0)),
        ],
        out_specs=[
            pl.BlockSpec((1, _TNA, _D), lambda n, r: (r, n, 0)),
            pl.BlockSpec((_TNA, 8), lambda n, r: (n, 0)),
            pl.BlockSpec((_TNA, _D), lambda n, r: (n, 0)),
            pl.BlockSpec((1, 8), lambda n, r: (0, 0)),
        ],
        out_shape=[
            jax.ShapeDtypeStruct((_R, _N, _D), jnp.float32),
            jax.ShapeDtypeStruct((_N, 8), jnp.float32),
            jax.ShapeDtypeStruct((_N, _D), jnp.float32),
            jax.ShapeDtypeStruct((1, 8), jnp.float32),
        ],
    )(x, xbf, weight, p, self_w, self_b2)


_sc_mesh = plsc.VectorSubcoreMesh(core_axis_name="c", subcore_axis_name="s")


@functools.partial(
    pl.kernel,
    out_type=jax.ShapeDtypeStruct((2, 2, _AGGR, _DH), jnp.float32),
    mesh=_sc_mesh,
    compiler_params=pltpu.CompilerParams(needs_layout_passes=False),
    scratch_types=[
        pltpu.VMEM((_NROW,), jnp.float32),
        pltpu.VMEM((_NROW,), jnp.float32),
        pltpu.VMEM((128, 128), jnp.float32),
        pltpu.VMEM((4 * _BLK,), jnp.int32),
        pltpu.VMEM((_CB, _DH), jnp.float32),
        pltpu.VMEM((_CB, _DH), jnp.float32),
        pltpu.VMEM((_CB,), jnp.int32),
        pltpu.VMEM((_CB,), jnp.int32),
        pltpu.VMEM((_CB,), jnp.int32),
        pltpu.VMEM((_CB,), jnp.int32),
        pltpu.VMEM((_CB,), jnp.float32),
        pltpu.VMEM((_CB,), jnp.float32),
        pltpu.VMEM((16,), jnp.float32),
        pltpu.VMEM((1, 128), jnp.int32),
        pltpu.VMEM((2 * _NBLK + 16,), jnp.int32),
        pltpu.VMEM_SHARED((128, 128), jnp.float32),
        pltpu.VMEM_SHARED((_AGGR, _DH), jnp.float32),
        pltpu.SemaphoreType.DMA,
        pltpu.SemaphoreType.DMA,
        pltpu.SemaphoreType.DMA,
        pltpu.SemaphoreType.DMA,
    ],
)
def _sc_agg(s1_h, s2_h, ed_h, mh_h, fl_h, xw_h, out_h,
            s1_v, s2_v, esum_v, edA,
            rows_v0, rows_v1, gidx_v0, gidx_v1, sidx_v0, sidx_v1,
            coef_v0, coef_v1, mh_v, ridx_v, flag_v,
            esum_sh, agg_sh, gsem0, gsem1, ssem0, ssem1):
    c = lax.axis_index("c")
    s = lax.axis_index("s")
    zero16 = jnp.zeros((16,), jnp.float32)

    def _load_blk(b):
        pltpu.sync_copy(ed_h.at[pl.ds((s * _NBLK + b) * 4 * _BLK, 4 * _BLK)],
                        edA)

    def _srcdst(sl16):
        src16 = edA[pl.ds(sl16, 16)]
        dst16 = edA[pl.ds(_BLK + sl16, 16)]
        return src16, dst16

    def _zt(i, _):
        sl = pl.ds(i * 16, 16)
        s1_v[sl] = zero16
        s2_v[sl] = zero16
        return 0
    lax.fori_loop(0, _NROW // 16, _zt, 0)

    def _ze(i, _):
        for v in range(8):
            esum_v[i, pl.ds(v * 16, 16)] = zero16
        return 0
    lax.fori_loop(0, 128, _ze, 0)

    pltpu.sync_copy(s1_h, s1_v.at[pl.ds(0, _N)])
    pltpu.sync_copy(s2_h, s2_v.at[pl.ds(0, _N)])
    pltpu.sync_copy(mh_h, mh_v)
    pltpu.sync_copy(fl_h.at[s], flag_v)

    for v in range(8):
        ridx_v[0, pl.ds(v * 16, 16)] = lax.iota(jnp.int32, 16) + v * 16

    def _zero_rows():
        def _zr(e, _):
            for v in range(_DH // 16):
                rows_v0[e, pl.ds(v * 16, 16)] = zero16
            return 0
        lax.fori_loop(0, _CB, _zr, 0)

    _zero_rows()

    r0 = s * _RPH

    def _zero_agg():
        for i in range(_RPH // _CB):
            pltpu.sync_copy(rows_v0, agg_sh.at[pl.ds(r0 + i * _CB, _CB)])

    @pl.when(s == 0)
    def _():
        pltpu.sync_copy(esum_v, esum_sh)

    plsc.subcore_barrier()

    mhat = mh_v[...]

    def _escore(src16, dst16):
        e = plsc.load_gather(s1_v, [src16]) + plsc.load_gather(s2_v, [dst16])
        e = jnp.where(e > 0, e, e * 0.01)
        return jnp.exp(e - mhat)

    def _esum_idx(dst16):
        return [lax.shift_right_logical(dst16, 7), jnp.bitwise_and(dst16, 127)]

    def _pa_blk(b, _):
        _load_blk(b)

        def _pa(i, _):
            src16, dst16 = _srcdst(i * 16)
            ex = _escore(src16, dst16)
            plsc.addupdate_scatter(esum_v, _esum_idx(dst16), ex)
            return 0
        lax.fori_loop(0, _BLK // 16, _pa, 0)
        return 0
    lax.fori_loop(0, _NBLK, _pa_blk, 0)

    pltpu.sync_copy(esum_v, esum_sh.at[ridx_v.at[0]], add=True)
    plsc.subcore_barrier()
    pltpu.sync_copy(esum_sh, esum_v)

    for h in range(2):
        _zero_rows()
        _zero_agg()
        plsc.subcore_barrier()

        bufs = ((rows_v0, gidx_v0, sidx_v0, coef_v0, gsem0, ssem0),
                (rows_v1, gidx_v1, sidx_v1, coef_v1, gsem1, ssem1))
        nch = _BLK // _CB

        def _build(u, st):
            rows, gidx, sidx, coef, gsem, ssem = st
            for v in range(_CB // 16):
                sl16 = u * _CB + v * 16
                osl = pl.ds(v * 16, 16)
                src16, dst16 = _srcdst(sl16)
                ex = _escore(src16, dst16)
                es = plsc.load_gather(esum_v, _esum_idx(dst16))
                nw16 = plsc.bitcast(edA[pl.ds(3 * _BLK + sl16, 16)],
                                    jnp.float32)
                cf = ex * nw16 / (es + 1e-9)
                t16 = dst16 - h * _NH
                inr = jnp.logical_and(t16 >= 0, t16 < _NH)
                coef[osl] = jnp.where(inr, cf, 0.0)
                sidx[osl] = jnp.where(inr, t16, _NH)
                gidx[osl] = edA[pl.ds(2 * _BLK + sl16, 16)] + c

        def _scale(st):
            rows, gidx, sidx, coef, gsem, ssem = st

            def _sc4(i4, _):
                for k in range(4):
                    e2 = i4 * 4 + k
                    cvec = plsc.load_gather(
                        coef, [jnp.zeros((16,), jnp.int32) + e2])
                    for v in range(_DH // 16):
                        csl = pl.ds(v * 16, 16)
                        rows[e2, csl] = rows[e2, csl] * cvec
                return 0
            lax.fori_loop(0, _CB // 4, _sc4, 0)

        def _pb_blk(b, _):
            fv = flag_v[pl.ds(h * _NBLK + b, 16)]

            @pl.when(fv[0] > 0)
            def _():
                _pb_blk_body(b)
            return 0

        def _pb_blk_body(b):
            _load_blk(b)

            _build(0, bufs[0])
            gd = {0: pltpu.async_copy(
                xw_h.at[bufs[0][1]], bufs[0][0], bufs[0][4])}
            sd = {}
            for u in range(nch):
                cur = bufs[u % 2]
                nxt = bufs[(u + 1) % 2]
                if u + 1 < nch:
                    if u >= 1:
                        sd[u - 1].wait()
                    _build(u + 1, nxt)
                    gd[u + 1] = pltpu.async_copy(
                        xw_h.at[nxt[1]], nxt[0], nxt[4])
                gd[u].wait()
                _scale(cur)
                sd[u] = pltpu.async_copy(
                    cur[0], agg_sh.at[cur[2]], cur[5], add=True)
            sd[nch - 2].wait()
            sd[nch - 1].wait()
        lax.fori_loop(0, _NBLK, _pb_blk, 0)

        plsc.subcore_barrier()
        pltpu.sync_copy(agg_sh.at[pl.ds(r0, _RPH)],
                        out_h.at[c, h, pl.ds(r0, _RPH)])
        plsc.subcore_barrier()


def _sg_body(q_ref, k_ref, ti_ref, tj_ref, xj_ref, g1_ref, so_ref,
             pw_ref, pb_ref, g2_ref, out_ref, num_ref, den_ref):
    j = pl.program_id(1)

    @pl.when(j == 0)
    def _():
        num_ref[...] = jnp.zeros_like(num_ref)
        den_ref[...] = jnp.zeros_like(den_ref)

    sc = q_ref[...] + k_ref[0]
    sc = jnp.where(sc > 0, sc, sc * 0.01)
    allowed = ti_ref[...] != tj_ref[0]
    w = jnp.where(allowed, jnp.exp(sc), 0.0)
    num_ref[...] += jnp.dot(w.astype(jnp.bfloat16), xj_ref[...],
                            preferred_element_type=jnp.float32)
    den_ref[...] += jnp.sum(w, axis=1, keepdims=True)

    @pl.when(j == _NT - 1)
    def _():
        sg_msg = num_ref[...] / den_ref[...]
        gate_in = g1_ref[...] + jnp.dot(
            sg_msg, g2_ref[...], preferred_element_type=jnp.float32)
        gate = 1.0 / (1.0 + jnp.exp(-gate_in))
        sg_out = (jnp.dot(sg_msg, pw_ref[...], preferred_element_type=jnp.float32)
                  + pb_ref[...]) * gate
        out_ref[...] = so_ref[...] + sg_out


def _run_sg(q2, k2, ti2, tj2, x, g12, so, pw, pb2, g2w):
    return pl.pallas_call(
        _sg_body,
        grid=(_NT, _NT),
        in_specs=[
            pl.BlockSpec((_TN, 1), lambda i, j: (i, 0)),
            pl.BlockSpec((1, 1, _TN), lambda i, j: (j, 0, 0)),
            pl.BlockSpec((_TN, 1), lambda i, j: (i, 0)),
            pl.BlockSpec((1, 1, _TN), lambda i, j: (j, 0, 0)),
            pl.BlockSpec((_TN, _D), lambda i, j: (j, 0)),
            pl.BlockSpec((_TN, 1), lambda i, j: (i, 0)),
            pl.BlockSpec((_TN, _D), lambda i, j: (i, 0)),
            pl.BlockSpec((_D, _D), lambda i, j: (0, 0)),
            pl.BlockSpec((1, _D), lambda i, j: (0, 0)),
            pl.BlockSpec((_D, 1), lambda i, j: (0, 0)),
        ],
        out_specs=pl.BlockSpec((_TN, _D), lambda i, j: (i, 0)),
        out_shape=jax.ShapeDtypeStruct((_N, _D), jnp.float32),
        scratch_shapes=[
            pltpu.VMEM((_TN, _D), jnp.float32),
            pltpu.VMEM((_TN, 1), jnp.float32),
        ],
    )(q2, k2, ti2, tj2, x, g12, so, pw, pb2, g2w)


def kernel(x, edge_index, rel_type, subgraph_type, norm, weight, attn_w,
           subgraph_attn_w, sg_proj_w, sg_proj_b, sg_gate_w, self_w, self_b):
    n, d = x.shape

    p = jnp.concatenate(
        [attn_w[:d], attn_w[d:], subgraph_attn_w[:d], subgraph_attn_w[d:],
         sg_gate_w[:d], jnp.zeros((d, 3), jnp.float32)], axis=1)

    xbf = x.astype(jnp.bfloat16)
    xw, sprj, selfout, stats = _run_proj(
        x, xbf, weight.astype(jnp.bfloat16), p, self_w, self_b.reshape(1, d))
    mh16 = jnp.full((16,), jnp.maximum(stats[0, 0] + stats[0, 1], 0.0),
                    jnp.float32)
    s1 = sprj[:, 0]
    s2 = sprj[:, 1]
    kk = sprj[:, 2]
    qq = sprj[:, 3]
    g1 = sprj[:, 4]

    src = edge_index[0]
    dst = edge_index[1]
    pad = _EPAD - _E
    srcp = jnp.concatenate([src, jnp.zeros((pad,), jnp.int32)])
    dstp = jnp.concatenate([dst, jnp.full((pad,), n, jnp.int32)])
    gb2 = (rel_type * n + src) * 2
    gbp = jnp.concatenate([gb2, jnp.zeros((pad,), jnp.int32)])
    nwp = jnp.concatenate([norm, jnp.zeros((pad,), jnp.float32)])

    order = jnp.argsort((dstp >= _NH).astype(jnp.int32), stable=True)

    def _bc(a):
        a = a[order]
        return a.reshape(_NBLK, _NSUB, _BLK).transpose(1, 0, 2).reshape(-1)

    srcp = _bc(srcp)
    dstp = _bc(dstp)
    gbp = _bc(gbp)
    nwp = _bc(nwp)
    dblk = dstp.reshape(_NSUB, _NBLK, _BLK)
    live = nwp.reshape(_NSUB, _NBLK, _BLK) != 0.0
    fl0 = jnp.any((dblk < _NH) & live, axis=2)
    fl1 = jnp.any((dblk >= _NH) & live, axis=2)
    flags = jnp.concatenate(
        [fl0, fl1, jnp.zeros((_NSUB, 16), bool)], axis=1).astype(jnp.int32)
    ed = jnp.stack(
        [srcp, dstp, gbp, lax.bitcast_convert_type(nwp, jnp.int32)], axis=0)
    ed = ed.reshape(4, _NSUB, _NBLK, _BLK).transpose(1, 2, 0, 3).reshape(-1)
    xwf = xw.reshape(_R * n * 2, _DH)

    agg4 = _sc_agg(s1, s2, ed, mh16, flags, xwf)

    sgso = _run_sg(
        qq.reshape(n, 1), kk.reshape(_NT, 1, _TN),
        subgraph_type.reshape(n, 1), subgraph_type.reshape(_NT, 1, _TN),
        xbf, g1.reshape(n, 1), selfout,
        sg_proj_w, sg_proj_b.reshape(1, d), sg_gate_w[d:])

    agg = jnp.concatenate(
        [jnp.concatenate([agg4[0, 0, :_NH], agg4[1, 0, :_NH]], axis=1),
         jnp.concatenate([agg4[0, 1, :_NH], agg4[1, 1, :_NH]], axis=1)],
        axis=0)[:n]
    return sgso + agg
